# baseline (speedup 1.0000x reference)
import functools

import jax
import jax.numpy as jnp
from jax import lax
from jax.experimental import pallas as pl
from jax.experimental.pallas import tpu as pltpu
from jax.experimental.pallas import tpu_sc as plsc

HASH_SIZE = 1000000
EMB = 16
F = 26
FI = 26
FW = F + FI
B = 16384
D_IN = F * EMB
NC, NS = 2, 16
NW = NC * NS
BPW = B // NW


def _sc_wide(sparse_idx, inter_idx, w_wide_flat):
    mesh = plsc.VectorSubcoreMesh(core_axis_name="c", subcore_axis_name="s")

    @functools.partial(
        pl.kernel,
        mesh=mesh,
        compiler_params=pltpu.CompilerParams(use_tc_tiling_on_sc=False),
        out_type=jax.ShapeDtypeStruct((B,), jnp.float32),
        scratch_types=[
            pltpu.VMEM((FW, BPW), jnp.int32),
            pltpu.VMEM((FW, BPW), jnp.float32),
            pltpu.VMEM((BPW,), jnp.float32),
            pltpu.SemaphoreType.DMA,
        ],
    )
    def kw(sidx_hbm, iidx_hbm, ww_hbm, wide_hbm, widx_v, wval_v, wacc_v,
           sem_w):
        wid = lax.axis_index("s") * NC + lax.axis_index("c")
        base = wid * BPW

        pltpu.sync_copy(sidx_hbm.at[pl.ds(0, F), pl.ds(base, BPW)],
                        widx_v.at[pl.ds(0, F)])
        pltpu.sync_copy(iidx_hbm.at[pl.ds(0, FI), pl.ds(base, BPW)],
                        widx_v.at[pl.ds(F, FI)])
        wide_copies = [
            pltpu.async_copy(ww_hbm.at[widx_v.at[j]], wval_v.at[j], sem_w)
            for j in range(FW)
        ]
        for c in wide_copies:
            c.wait()

        def acc_body(i, _):
            off = i * 16
            acc = wval_v[0, pl.ds(off, 16)]
            for j in range(1, FW):
                acc = acc + wval_v[j, pl.ds(off, 16)]
            wacc_v[pl.ds(off, 16)] = acc
            return 0

        lax.fori_loop(0, BPW // 16, acc_body, 0)
        pltpu.sync_copy(wacc_v, wide_hbm.at[pl.ds(base, BPW)])

    return kw(sparse_idx, inter_idx, w_wide_flat)


def _sc_deep(sparse_idx, embf):
    mesh = plsc.VectorSubcoreMesh(core_axis_name="c", subcore_axis_name="s")

    @functools.partial(
        pl.kernel,
        mesh=mesh,
        compiler_params=pltpu.CompilerParams(use_tc_tiling_on_sc=False),
        out_type=jax.ShapeDtypeStruct((B // 8, 4, 8, 128), jnp.float32),
        scratch_types=[
            pltpu.VMEM((F, BPW), jnp.int32),
            pltpu.VMEM((2, BPW * EMB), jnp.int32),
            pltpu.VMEM((2, BPW * EMB), jnp.float32),
            pltpu.VMEM((2, BPW // 8, 8, EMB), jnp.float32),
            pltpu.SemaphoreType.DMA,
            pltpu.SemaphoreType.DMA,
        ],
    )
    def kd(sidx_hbm, embf_hbm, xemb_hbm, didx_v, idx8k_v, rows_g, rows_s,
           sem_g, sem_s):
        wid = lax.axis_index("s") * NC + lax.axis_index("c")
        base = wid * BPW
        b8 = wid * (BPW // 8)

        pltpu.sync_copy(sidx_hbm.at[pl.ds(0, F), pl.ds(base, BPW)],
                        didx_v)

        coordoff = lax.broadcasted_iota(jnp.int32, (16,), 0) * HASH_SIZE

        def build_idx(f, b):
            def bb(g, _):
                sv = didx_v[f, pl.ds(g * 16, 16)]
                for u in range(16):
                    idx8k_v[b, pl.ds((g * 16 + u) * 16, 16)] = (
                        coordoff + sv[u])
                return 0
            lax.fori_loop(0, BPW // 16, bb, 0)

        def relabel(buf):
            def body(i, _):
                for r in range(8):
                    rows_s[buf, i, r, :] = rows_g[
                        buf, pl.ds((i * 8 + r) * 16, 16)]
                return 0
            lax.fori_loop(0, BPW // 8, body, 0)

        gathers = [None] * F
        stores = [None, None]
        build_idx(0, 0)
        gathers[0] = pltpu.async_copy(embf_hbm.at[idx8k_v.at[0]],
                                      rows_g.at[0], sem_g)
        for f in range(F):
            buf = f % 2
            if f + 1 < F:
                build_idx(f + 1, 1 - buf)
                gathers[f + 1] = pltpu.async_copy(
                    embf_hbm.at[idx8k_v.at[1 - buf]], rows_g.at[1 - buf],
                    sem_g)
            gathers[f].wait()
            if stores[buf] is not None:
                stores[buf].wait()
            relabel(buf)
            fq, c0 = (EMB * f) // 128, (EMB * f) % 128
            stores[buf] = pltpu.async_copy(
                rows_s.at[buf],
                xemb_hbm.at[pl.ds(b8, BPW // 8), fq, pl.ds(0, 8),
                            pl.ds(c0, EMB)],
                sem_s)

        stores[0].wait()
        stores[1].wait()

    return kd(sparse_idx, embf)


def _tc_mlp(x4d, wide, W1p, b1, W2, b2, w_out, b_out):
    BT = 2048
    H1, H2 = W1p.shape[1], W2.shape[1]

    def body(x_ref, wide_ref, W1_ref, b1_ref, W2_ref, b2_ref, wo_ref,
             bo_ref, o_ref):
        x = x_ref[...]
        lanes = lax.broadcasted_iota(jnp.int32, (BT, 128), 1)
        W1 = W1_ref[...]
        h = jnp.zeros((BT, H1), jnp.float32) + b1_ref[...]
        for q in range(4):
            xq = x[:, q].reshape(BT, 128)
            if q == 3:
                xq = jnp.where(lanes < D_IN - 3 * 128, xq, 0.0)
            h = h + jnp.dot(xq, W1[128 * q:128 * (q + 1), :],
                            preferred_element_type=jnp.float32)
        h = jnp.maximum(h, 0.0)
        h = jnp.dot(h, W2_ref[...],
                    preferred_element_type=jnp.float32) + b2_ref[...]
        h = jnp.maximum(h, 0.0)
        z = jnp.dot(h, wo_ref[...], preferred_element_type=jnp.float32)
        z = z + wide_ref[...] + bo_ref[...]
        o_ref[...] = jax.nn.sigmoid(z)

    return pl.pallas_call(
        body,
        grid=(B // BT,),
        in_specs=[
            pl.BlockSpec((BT // 8, 4, 8, 128), lambda i: (i, 0, 0, 0)),
            pl.BlockSpec((BT, 1), lambda i: (i, 0)),
            pl.BlockSpec((512, H1), lambda i: (0, 0)),
            pl.BlockSpec((1, H1), lambda i: (0, 0)),
            pl.BlockSpec((H1, H2), lambda i: (0, 0)),
            pl.BlockSpec((1, H2), lambda i: (0, 0)),
            pl.BlockSpec((H2, 1), lambda i: (0, 0)),
            pl.BlockSpec((1, 1), lambda i: (0, 0)),
        ],
        out_specs=pl.BlockSpec((BT, 1), lambda i: (i, 0)),
        out_shape=jax.ShapeDtypeStruct((B, 1), jnp.float32),
    )(x4d, wide, W1p, b1.reshape(1, H1), W2, b2.reshape(1, H2), w_out,
      b_out.reshape(1, 1))


def kernel(sparse_idx, inter_idx, emb_v, w_wide, W1, b1, W2, b2, w_out,
           b_out):
    wide_sum = _sc_wide(sparse_idx, inter_idx, w_wide.reshape(-1))
    x4d = _sc_deep(sparse_idx, emb_v.T.reshape(-1))
    W1p = jnp.zeros((512, W1.shape[1]), jnp.float32).at[:D_IN].set(W1)
    return _tc_mlp(x4d, wide_sum.reshape(B, 1), W1p, b1, W2, b2, w_out,
                   b_out)


# speedup vs baseline: 3.0088x; 3.0088x over previous
import functools

import jax
import jax.numpy as jnp
from jax import lax
from jax.experimental import pallas as pl
from jax.experimental.pallas import tpu as pltpu
from jax.experimental.pallas import tpu_sc as plsc

HASH_SIZE = 1000000
EMB = 16
F = 26
FI = 26
FW = F + FI
B = 16384
D_IN = F * EMB
NC, NS = 2, 16
NW = NC * NS
BPW = B // NW


def _sc_wide(sparse_idx, inter_idx, w_wide_flat):
    mesh = plsc.VectorSubcoreMesh(core_axis_name="c", subcore_axis_name="s")

    @functools.partial(
        pl.kernel,
        mesh=mesh,
        compiler_params=pltpu.CompilerParams(use_tc_tiling_on_sc=False),
        out_type=jax.ShapeDtypeStruct((B,), jnp.float32),
        scratch_types=[
            pltpu.VMEM((FW, BPW), jnp.int32),
            pltpu.VMEM((FW, BPW), jnp.float32),
            pltpu.VMEM((BPW,), jnp.float32),
            pltpu.SemaphoreType.DMA,
        ],
    )
    def kw(sidx_hbm, iidx_hbm, ww_hbm, wide_hbm, widx_v, wval_v, wacc_v,
           sem_w):
        wid = lax.axis_index("s") * NC + lax.axis_index("c")
        base = wid * BPW

        pltpu.sync_copy(sidx_hbm.at[pl.ds(0, F), pl.ds(base, BPW)],
                        widx_v.at[pl.ds(0, F)])
        pltpu.sync_copy(iidx_hbm.at[pl.ds(0, FI), pl.ds(base, BPW)],
                        widx_v.at[pl.ds(F, FI)])
        wide_copies = [
            pltpu.async_copy(ww_hbm.at[widx_v.at[j]], wval_v.at[j], sem_w)
            for j in range(FW)
        ]
        for c in wide_copies:
            c.wait()

        def acc_body(i, _):
            off = i * 16
            acc = wval_v[0, pl.ds(off, 16)]
            for j in range(1, FW):
                acc = acc + wval_v[j, pl.ds(off, 16)]
            wacc_v[pl.ds(off, 16)] = acc
            return 0

        lax.fori_loop(0, BPW // 16, acc_body, 0)
        pltpu.sync_copy(wacc_v, wide_hbm.at[pl.ds(base, BPW)])

    return kw(sparse_idx, inter_idx, w_wide_flat)


def _sc_deep(sparse_idx, emb_v2d):
    mesh = plsc.VectorSubcoreMesh(core_axis_name="c", subcore_axis_name="s")

    @functools.partial(
        pl.kernel,
        mesh=mesh,
        compiler_params=pltpu.CompilerParams(use_tc_tiling_on_sc=False),
        out_type=jax.ShapeDtypeStruct((B // 8, 4, 8, 128), jnp.float32),
        scratch_types=[
            pltpu.VMEM((F, BPW), jnp.int32),
            pltpu.VMEM((2, BPW, EMB), jnp.float32),
            pltpu.VMEM((2, BPW // 8, 8, EMB), jnp.float32),
            pltpu.SemaphoreType.DMA,
            pltpu.SemaphoreType.DMA,
        ],
    )
    def kd(sidx_hbm, emb_hbm, xemb_hbm, didx_v, rows_g, rows_s, sem_g,
           sem_s):
        wid = lax.axis_index("s") * NC + lax.axis_index("c")
        base = wid * BPW
        b8 = wid * (BPW // 8)

        pltpu.sync_copy(sidx_hbm.at[pl.ds(0, F), pl.ds(base, BPW)],
                        didx_v)

        def relabel(buf):
            def body(i, _):
                for r in range(8):
                    rows_s[buf, i, r, :] = rows_g[buf, i * 8 + r, :]
                return 0
            lax.fori_loop(0, BPW // 8, body, 0)

        gathers = [None] * F
        stores = [None, None]
        gathers[0] = pltpu.async_copy(emb_hbm.at[didx_v.at[0]],
                                      rows_g.at[0], sem_g)
        for f in range(F):
            buf = f % 2
            gathers[f].wait()
            if f + 1 < F:
                gathers[f + 1] = pltpu.async_copy(
                    emb_hbm.at[didx_v.at[f + 1]], rows_g.at[1 - buf],
                    sem_g)
            if stores[buf] is not None:
                stores[buf].wait()
            relabel(buf)
            fq, c0 = (EMB * f) // 128, (EMB * f) % 128
            stores[buf] = pltpu.async_copy(
                rows_s.at[buf],
                xemb_hbm.at[pl.ds(b8, BPW // 8), fq, pl.ds(0, 8),
                            pl.ds(c0, EMB)],
                sem_s)

        stores[0].wait()
        stores[1].wait()

    return kd(sparse_idx, emb_v2d)


def _tc_mlp(x4d, wide, W1p, b1, W2, b2, w_out, b_out):
    BT = 2048
    H1, H2 = W1p.shape[1], W2.shape[1]

    def body(x_ref, wide_ref, W1_ref, b1_ref, W2_ref, b2_ref, wo_ref,
             bo_ref, o_ref):
        x = x_ref[...]
        lanes = lax.broadcasted_iota(jnp.int32, (BT, 128), 1)
        W1 = W1_ref[...]
        h = jnp.zeros((BT, H1), jnp.float32) + b1_ref[...]
        for q in range(4):
            xq = x[:, q].reshape(BT, 128)
            if q == 3:
                xq = jnp.where(lanes < D_IN - 3 * 128, xq, 0.0)
            h = h + jnp.dot(xq, W1[128 * q:128 * (q + 1), :],
                            preferred_element_type=jnp.float32)
        h = jnp.maximum(h, 0.0)
        h = jnp.dot(h, W2_ref[...],
                    preferred_element_type=jnp.float32) + b2_ref[...]
        h = jnp.maximum(h, 0.0)
        z = jnp.dot(h, wo_ref[...], preferred_element_type=jnp.float32)
        z = z + wide_ref[...] + bo_ref[...]
        o_ref[...] = jax.nn.sigmoid(z)

    return pl.pallas_call(
        body,
        grid=(B // BT,),
        in_specs=[
            pl.BlockSpec((BT // 8, 4, 8, 128), lambda i: (i, 0, 0, 0)),
            pl.BlockSpec((BT, 1), lambda i: (i, 0)),
            pl.BlockSpec((512, H1), lambda i: (0, 0)),
            pl.BlockSpec((1, H1), lambda i: (0, 0)),
            pl.BlockSpec((H1, H2), lambda i: (0, 0)),
            pl.BlockSpec((1, H2), lambda i: (0, 0)),
            pl.BlockSpec((H2, 1), lambda i: (0, 0)),
            pl.BlockSpec((1, 1), lambda i: (0, 0)),
        ],
        out_specs=pl.BlockSpec((BT, 1), lambda i: (i, 0)),
        out_shape=jax.ShapeDtypeStruct((B, 1), jnp.float32),
    )(x4d, wide, W1p, b1.reshape(1, H1), W2, b2.reshape(1, H2), w_out,
      b_out.reshape(1, 1))


def kernel(sparse_idx, inter_idx, emb_v, w_wide, W1, b1, W2, b2, w_out,
           b_out):
    wide_sum = _sc_wide(sparse_idx, inter_idx, w_wide.reshape(-1))
    x4d = _sc_deep(sparse_idx, emb_v)
    W1p = jnp.zeros((512, W1.shape[1]), jnp.float32).at[:D_IN].set(W1)
    return _tc_mlp(x4d, wide_sum.reshape(B, 1), W1p, b1, W2, b2, w_out,
                   b_out)
